# Initial kernel scaffold; baseline (speedup 1.0000x reference)
import jax
import jax.numpy as jnp
from jax.experimental import pallas as pl


def kernel(position, pe):
    raise NotImplementedError("write your pallas kernel here")


# speedup vs baseline: 4.8638x; 4.8638x over previous
import functools

import jax
import jax.numpy as jnp
from jax import lax
from jax.experimental import pallas as pl
from jax.experimental.pallas import tpu as pltpu
from jax.experimental.pallas import tpu_sc as plsc

_EMBED = 128
_BATCH = 32
_SEQ = 8192
_B = _BATCH * _SEQ

_NC = 2
_NS = 16
_NW = _NC * _NS
_PER_W = _B // _NW
_CHUNK = 128
_NCHUNK = _PER_W // _CHUNK

_gather_rows_cache = None


def _build():
    global _gather_rows_cache
    if _gather_rows_cache is not None:
        return _gather_rows_cache

    mesh = plsc.VectorSubcoreMesh(core_axis_name="c", subcore_axis_name="s")

    @functools.partial(
        pl.kernel,
        mesh=mesh,
        out_type=jax.ShapeDtypeStruct((_B, _EMBED), jnp.float32),
        scratch_types=[
            pltpu.VMEM((_CHUNK,), jnp.int32),
            pltpu.VMEM((_CHUNK, _EMBED), jnp.float32),
            pltpu.SemaphoreType.DMA,
        ],
    )
    def _gather_rows(table_hbm, idx_hbm, out_hbm, idx_v, rows_v, sem):
        wid = lax.axis_index("s") * _NC + lax.axis_index("c")
        base = wid * _PER_W

        def body(i, carry):
            off = base + i * _CHUNK
            pltpu.sync_copy(idx_hbm.at[pl.ds(off, _CHUNK)], idx_v)
            pltpu.async_copy(table_hbm.at[idx_v], rows_v, sem).wait()
            pltpu.sync_copy(rows_v, out_hbm.at[pl.ds(off, _CHUNK)])
            return carry

        lax.fori_loop(0, _NCHUNK, body, 0)

    _gather_rows_cache = _gather_rows
    return _gather_rows


def kernel(position, pe):
    idx = position.reshape(_B)
    out = _build()(pe, idx)
    return out.reshape(_BATCH, _SEQ, _EMBED)


# speedup vs baseline: 7.8553x; 1.6150x over previous
import functools

import jax
import jax.numpy as jnp
from jax import lax
from jax.experimental import pallas as pl
from jax.experimental.pallas import tpu as pltpu
from jax.experimental.pallas import tpu_sc as plsc

_EMBED = 128
_BATCH = 32
_SEQ = 8192
_B = _BATCH * _SEQ

_NC = 2
_NS = 16
_NW = _NC * _NS
_PER_W = _B // _NW
_CHUNK = 128
_NCHUNK = _PER_W // _CHUNK

_gather_rows_cache = None


def _build():
    global _gather_rows_cache
    if _gather_rows_cache is not None:
        return _gather_rows_cache

    mesh = plsc.VectorSubcoreMesh(core_axis_name="c", subcore_axis_name="s")

    @functools.partial(
        pl.kernel,
        mesh=mesh,
        out_type=jax.ShapeDtypeStruct((_B, _EMBED), jnp.float32),
        scratch_types=[
            pltpu.VMEM((_NCHUNK, _CHUNK), jnp.int32),
            pltpu.VMEM((_CHUNK, _EMBED), jnp.float32),
            pltpu.VMEM((_CHUNK, _EMBED), jnp.float32),
            pltpu.SemaphoreType.DMA,
            pltpu.SemaphoreType.DMA,
            pltpu.SemaphoreType.DMA,
            pltpu.SemaphoreType.DMA,
        ],
    )
    def _gather_rows(table_hbm, idx_hbm, out_hbm, idx_v, rows_a, rows_b,
                     gsem_a, gsem_b, wsem_a, wsem_b):
        wid = lax.axis_index("s") * _NC + lax.axis_index("c")
        cbase = wid * _NCHUNK

        bufs = (rows_a, rows_b)
        gsems = (gsem_a, gsem_b)
        wsems = (wsem_a, wsem_b)

        pltpu.sync_copy(idx_hbm.at[pl.ds(cbase, _NCHUNK)], idx_v)

        for b in range(2):
            pltpu.async_copy(table_hbm.at[idx_v.at[b]], bufs[b], gsems[b])

        def body(j, carry):
            for b in range(2):
                cur = 2 * j + b
                pltpu.make_async_copy(
                    table_hbm.at[idx_v.at[cur]], bufs[b], gsems[b]).wait()
                out_slice = out_hbm.at[pl.ds((cbase + cur) * _CHUNK, _CHUNK)]
                pltpu.async_copy(bufs[b], out_slice, wsems[b])
                pltpu.make_async_copy(bufs[b], out_slice, wsems[b]).wait()
                pltpu.async_copy(
                    table_hbm.at[idx_v.at[cur + 2]], bufs[b], gsems[b])
            return carry

        lax.fori_loop(0, (_NCHUNK - 2) // 2, body, 0)

        for b in range(2):
            cur = _NCHUNK - 2 + b
            pltpu.make_async_copy(
                table_hbm.at[idx_v.at[cur]], bufs[b], gsems[b]).wait()
            out_slice = out_hbm.at[pl.ds((cbase + cur) * _CHUNK, _CHUNK)]
            pltpu.async_copy(bufs[b], out_slice, wsems[b])
            pltpu.make_async_copy(bufs[b], out_slice, wsems[b]).wait()

    _gather_rows_cache = _gather_rows
    return _gather_rows


def kernel(position, pe):
    idx = position.reshape(_B // _CHUNK, _CHUNK)
    out = _build()(pe, idx)
    return out.reshape(_BATCH, _SEQ, _EMBED)
